# baseline (speedup 1.0000x reference)
import jax
import jax.numpy as jnp
from jax import lax
from jax.experimental import pallas as pl
from jax.experimental.pallas import tpu as pltpu
from jax.experimental.pallas import tpu_sc as plsc

N = 10000
D = 128
E = 320000
EPS = 1e-5

NC = 2
NS = 16
NW = NC * NS
EPW = E // NW
CHUNK = 80
NCH = EPW // CHUNK
BLK = 80
NBLK = N // BLK

_LANES = 16


def _sc_agg_body(h_hbm, comb_hbm, ew_hbm, out_hbm,
                 combv, eww, srcW0, dstW0, srcW1, dstW1, rows, rows1,
                 acc, sem0, sem1, sems0, sems1):
    c = lax.axis_index("c")
    s = lax.axis_index("s")
    wid = c * NS + s

    pltpu.sync_copy(comb_hbm.at[wid], combv)
    pltpu.sync_copy(ew_hbm.at[wid], eww)

    mask14 = jnp.full((_LANES,), (1 << 14) - 1, jnp.int32)
    sh14 = jnp.full((_LANES,), 14, jnp.int32)
    zero16 = jnp.zeros((_LANES,), jnp.float32)

    def _zrow(i, carry):
        for j in range(D // _LANES):
            rows[i, pl.ds(j * _LANES, _LANES)] = zero16
        return carry

    lax.fori_loop(0, CHUNK, _zrow, 0)
    for t in range((NBLK + NS - 1) // NS):
        blk = s + NS * t

        @pl.when(blk < NBLK)
        def _():
            pltpu.sync_copy(rows, acc.at[pl.ds(blk * BLK, BLK)])
    plsc.subcore_barrier()

    def _decode(i, sW, dW):
        for g in range(CHUNK // _LANES):
            cv = combv[i, pl.ds(g * _LANES, _LANES)]
            sW[pl.ds(g * _LANES, _LANES)] = cv & mask14
            dW[pl.ds(g * _LANES, _LANES)] = lax.shift_right_logical(cv, sh14)

    def _scale(i, rbuf):
        def _group(g, c2):
            w16 = eww[pl.ds(i * CHUNK + g * _LANES, _LANES)]
            for e in range(_LANES):
                w = jnp.take_along_axis(
                    w16, jnp.full((_LANES,), e, jnp.int32), axis=0,
                    mode="promise_in_bounds")
                r = g * _LANES + e
                for j in range(D // _LANES):
                    rbuf[r, pl.ds(j * _LANES, _LANES)] = (
                        rbuf[r, pl.ds(j * _LANES, _LANES)] * w)
            return c2

        lax.fori_loop(0, CHUNK // _LANES, _group, 0)

    _decode(0, srcW0, dstW0)
    pltpu.async_copy(h_hbm.at[srcW0], rows, sem0)
    _decode(1, srcW1, dstW1)
    pltpu.async_copy(h_hbm.at[srcW1], rows1, sem1)
    pltpu.make_async_copy(h_hbm.at[srcW0], rows, sem0).wait()
    _scale(0, rows)
    pltpu.async_copy(rows, acc.at[dstW0], sems0, add=True)
    pltpu.make_async_copy(h_hbm.at[srcW1], rows1, sem1).wait()
    _scale(1, rows1)
    pltpu.async_copy(rows1, acc.at[dstW1], sems1, add=True)
    pltpu.make_async_copy(rows, acc.at[dstW0], sems0).wait()
    _decode(2, srcW0, dstW0)
    pltpu.async_copy(h_hbm.at[srcW0], rows, sem0)

    def _pair(t, carry):
        i0 = 2 * t
        pltpu.make_async_copy(rows1, acc.at[dstW1], sems1).wait()
        _decode(i0 + 1, srcW1, dstW1)
        pltpu.async_copy(h_hbm.at[srcW1], rows1, sem1)
        pltpu.make_async_copy(h_hbm.at[srcW0], rows, sem0).wait()
        _scale(i0, rows)
        pltpu.async_copy(rows, acc.at[dstW0], sems0, add=True)
        pltpu.make_async_copy(h_hbm.at[srcW1], rows1, sem1).wait()
        _scale(i0 + 1, rows1)
        pltpu.async_copy(rows1, acc.at[dstW1], sems1, add=True)
        pltpu.make_async_copy(rows, acc.at[dstW0], sems0).wait()
        _decode(i0 + 2, srcW0, dstW0)
        pltpu.async_copy(h_hbm.at[srcW0], rows, sem0)
        return carry

    lax.fori_loop(1, (NCH - 1) // 2, _pair, 0)
    pltpu.make_async_copy(rows1, acc.at[dstW1], sems1).wait()
    pltpu.make_async_copy(h_hbm.at[srcW0], rows, sem0).wait()
    _scale(NCH - 1, rows)
    pltpu.sync_copy(rows, acc.at[dstW0], add=True)
    plsc.subcore_barrier()

    for t in range((NBLK + NS - 1) // NS):
        blk = s + NS * t

        @pl.when(blk < NBLK)
        def _():
            r0 = blk * BLK
            pltpu.sync_copy(acc.at[pl.ds(r0, BLK)],
                            out_hbm.at[c, pl.ds(r0, BLK)])


_sc_agg = pl.kernel(
    _sc_agg_body,
    out_type=jax.ShapeDtypeStruct((NC, N, D), jnp.float32),
    mesh=plsc.VectorSubcoreMesh(
        core_axis_name="c", subcore_axis_name="s",
        num_cores=NC, num_subcores=NS),
    scratch_types=[
        pltpu.VMEM((NCH, CHUNK), jnp.int32),
        pltpu.VMEM((EPW,), jnp.float32),
        pltpu.VMEM((CHUNK,), jnp.int32),
        pltpu.VMEM((CHUNK,), jnp.int32),
        pltpu.VMEM((CHUNK,), jnp.int32),
        pltpu.VMEM((CHUNK,), jnp.int32),
        pltpu.VMEM((CHUNK, D), jnp.float32),
        pltpu.VMEM((CHUNK, D), jnp.float32),
        pltpu.VMEM_SHARED((N, D), jnp.float32),
        pltpu.SemaphoreType.DMA,
        pltpu.SemaphoreType.DMA,
        pltpu.SemaphoreType.DMA,
        pltpu.SemaphoreType.DMA,
    ],
)


_ROW_BLK = 1000
_GRID = N // _ROW_BLK


def _tc_in_body(x_ref, w_ref, b_ref, o_ref):
    o_ref[...] = jnp.maximum(
        jnp.dot(x_ref[...], w_ref[...], preferred_element_type=jnp.float32)
        + b_ref[...], 0.0)


def _tc_in(x, W, b):
    return pl.pallas_call(
        _tc_in_body,
        grid=(_GRID,),
        in_specs=[
            pl.BlockSpec((_ROW_BLK, D), lambda i: (i, 0)),
            pl.BlockSpec((D, D), lambda i: (0, 0)),
            pl.BlockSpec((1, D), lambda i: (0, 0)),
        ],
        out_specs=pl.BlockSpec((_ROW_BLK, D), lambda i: (i, 0)),
        out_shape=jax.ShapeDtypeStruct((N, D), jnp.float32),
    )(x, W, b)


def _tc_mm_body(p_ref, h_ref, wr_ref, br_ref, wo_ref, t_ref, s_ref, q_ref):
    agg = p_ref[0] + p_ref[1]
    t = (jnp.dot(agg, wr_ref[...], preferred_element_type=jnp.float32)
         + jnp.dot(h_ref[...], wo_ref[...], preferred_element_type=jnp.float32)
         + br_ref[...])
    t_ref[...] = t

    @pl.when(pl.program_id(0) == 0)
    def _():
        s_ref[...] = jnp.zeros_like(s_ref)
        q_ref[...] = jnp.zeros_like(q_ref)

    s_ref[...] += jnp.sum(t, axis=0, keepdims=True)
    q_ref[...] += jnp.sum(t * t, axis=0, keepdims=True)


def _tc_mm(parts, h, W_rel, b_rel, W_root):
    return pl.pallas_call(
        _tc_mm_body,
        grid=(_GRID,),
        in_specs=[
            pl.BlockSpec((NC, _ROW_BLK, D), lambda i: (0, i, 0)),
            pl.BlockSpec((_ROW_BLK, D), lambda i: (i, 0)),
            pl.BlockSpec((D, D), lambda i: (0, 0)),
            pl.BlockSpec((1, D), lambda i: (0, 0)),
            pl.BlockSpec((D, D), lambda i: (0, 0)),
        ],
        out_specs=[
            pl.BlockSpec((_ROW_BLK, D), lambda i: (i, 0)),
            pl.BlockSpec((1, D), lambda i: (0, 0)),
            pl.BlockSpec((1, D), lambda i: (0, 0)),
        ],
        out_shape=[
            jax.ShapeDtypeStruct((N, D), jnp.float32),
            jax.ShapeDtypeStruct((1, D), jnp.float32),
            jax.ShapeDtypeStruct((1, D), jnp.float32),
        ],
    )(parts, h, W_rel, b_rel, W_root)


def _tc_bn_body(t_ref, s_ref, q_ref, g_ref, b_ref, f_ref, o_ref):
    mean = s_ref[...] / N
    var = q_ref[...] / N - mean * mean
    scale = g_ref[...] * lax.rsqrt(var + EPS)
    y = (t_ref[...] - mean) * scale + b_ref[...]
    o_ref[...] = jnp.where(f_ref[0, 0] > 0, jnp.maximum(y, 0.0), y)


def _tc_bn(t, ssum, sqsum, gamma, beta, relu_flag):
    return pl.pallas_call(
        _tc_bn_body,
        grid=(_GRID,),
        in_specs=[
            pl.BlockSpec((_ROW_BLK, D), lambda i: (i, 0)),
            pl.BlockSpec((1, D), lambda i: (0, 0)),
            pl.BlockSpec((1, D), lambda i: (0, 0)),
            pl.BlockSpec((1, D), lambda i: (0, 0)),
            pl.BlockSpec((1, D), lambda i: (0, 0)),
            pl.BlockSpec((1, 1), lambda i: (0, 0)),
        ],
        out_specs=pl.BlockSpec((_ROW_BLK, D), lambda i: (i, 0)),
        out_shape=jax.ShapeDtypeStruct((N, D), jnp.float32),
    )(t, ssum, sqsum, gamma, beta, relu_flag)


def kernel(x, adj, features, W_in, b_in, W_rel1, b_rel1, W_root1,
           W_rel2, b_rel2, W_root2, gamma1, beta1):
    comb = ((adj[1] << 14) | adj[0]).reshape(NW, NCH, CHUNK)
    ew = features.reshape(NW, EPW)
    b_in_r = b_in.reshape(1, D)
    gamma_r = gamma1.reshape(1, D)
    beta_r = beta1.reshape(1, D)

    h = _tc_in(x, W_in, b_in_r)

    W_rels = jnp.stack([W_rel1, W_rel2])
    b_rels = jnp.stack([b_rel1.reshape(1, D), b_rel2.reshape(1, D)])
    W_roots = jnp.stack([W_root1, W_root2])
    flags = jnp.array([[[1.0]], [[0.0]]], jnp.float32)

    n_layers = lax.optimization_barrier(jnp.int32(2))

    def _cond(carry):
        i, _ = carry
        return i < n_layers

    def _layer(carry):
        i, hc = carry
        wr = lax.dynamic_index_in_dim(W_rels, i, keepdims=False)
        br = lax.dynamic_index_in_dim(b_rels, i, keepdims=False)
        wroot = lax.dynamic_index_in_dim(W_roots, i, keepdims=False)
        flag = lax.dynamic_index_in_dim(flags, i, keepdims=False)
        parts = _sc_agg(hc, comb, ew)
        t, ssum, qsum = _tc_mm(parts, hc, wr, br, wroot)
        return i + 1, _tc_bn(t, ssum, qsum, gamma_r, beta_r, flag)

    _, out = lax.while_loop(_cond, _layer, (jnp.int32(0), h))
    return out


# speedup vs baseline: 1.1018x; 1.1018x over previous
import jax
import jax.numpy as jnp
from jax import lax
from jax.experimental import pallas as pl
from jax.experimental.pallas import tpu as pltpu
from jax.experimental.pallas import tpu_sc as plsc

N = 10000
D = 128
E = 320000
EPS = 1e-5

NC = 2
NS = 16
NW = NC * NS
EPW = E // NW
CHUNK = 80
NCH = EPW // CHUNK
BLK = 80
NBLK = N // BLK

_LANES = 16


def _sc_agg_body(h_hbm, comb_hbm, ew_hbm, out_hbm,
                 combv, eww, srcW0, dstW0, srcW1, dstW1, rows, rows1,
                 acc, sem0, sem1, sems0, sems1):
    c = lax.axis_index("c")
    s = lax.axis_index("s")
    wid = c * NS + s

    pltpu.sync_copy(comb_hbm.at[wid], combv)
    pltpu.sync_copy(ew_hbm.at[wid], eww)

    mask14 = jnp.full((_LANES,), (1 << 14) - 1, jnp.int32)
    sh14 = jnp.full((_LANES,), 14, jnp.int32)
    zero16 = jnp.zeros((_LANES,), jnp.float32)

    def _zrow(i, carry):
        for j in range(D // _LANES):
            rows[i, pl.ds(j * _LANES, _LANES)] = zero16
        return carry

    lax.fori_loop(0, CHUNK, _zrow, 0)
    for t in range((NBLK + NS - 1) // NS):
        blk = s + NS * t

        @pl.when(blk < NBLK)
        def _():
            pltpu.sync_copy(rows, acc.at[pl.ds(blk * BLK, BLK)])
    plsc.subcore_barrier()

    def _decode(i, sW, dW):
        for g in range(CHUNK // _LANES):
            cv = combv[i, pl.ds(g * _LANES, _LANES)]
            sW[pl.ds(g * _LANES, _LANES)] = cv & mask14
            dW[pl.ds(g * _LANES, _LANES)] = lax.shift_right_logical(cv, sh14)

    def _scale(i, rbuf):
        def _group(g, c2):
            w16 = eww[pl.ds(i * CHUNK + g * _LANES, _LANES)]
            for e in range(_LANES):
                w = jnp.take_along_axis(
                    w16, jnp.full((_LANES,), e, jnp.int32), axis=0,
                    mode="promise_in_bounds")
                r = g * _LANES + e
                for j in range(D // _LANES):
                    rbuf[r, pl.ds(j * _LANES, _LANES)] = (
                        rbuf[r, pl.ds(j * _LANES, _LANES)] * w)
            return c2

        lax.fori_loop(0, CHUNK // _LANES, _group, 0)

    _decode(0, srcW0, dstW0)
    pltpu.async_copy(h_hbm.at[srcW0], rows, sem0)

    def _pair(t, carry):
        i0 = 2 * t
        _decode(i0 + 1, srcW1, dstW1)
        pltpu.async_copy(h_hbm.at[srcW1], rows1, sem1)
        pltpu.make_async_copy(h_hbm.at[srcW0], rows, sem0).wait()
        _scale(i0, rows)
        pltpu.sync_copy(rows, acc.at[dstW0], add=True)
        _decode(i0 + 2, srcW0, dstW0)
        pltpu.async_copy(h_hbm.at[srcW0], rows, sem0)
        pltpu.make_async_copy(h_hbm.at[srcW1], rows1, sem1).wait()
        _scale(i0 + 1, rows1)
        pltpu.sync_copy(rows1, acc.at[dstW1], add=True)
        return carry

    lax.fori_loop(0, (NCH - 1) // 2, _pair, 0)
    pltpu.make_async_copy(h_hbm.at[srcW0], rows, sem0).wait()
    _scale(NCH - 1, rows)
    pltpu.sync_copy(rows, acc.at[dstW0], add=True)
    plsc.subcore_barrier()

    for t in range((NBLK + NS - 1) // NS):
        blk = s + NS * t

        @pl.when(blk < NBLK)
        def _():
            r0 = blk * BLK
            pltpu.sync_copy(acc.at[pl.ds(r0, BLK)],
                            out_hbm.at[c, pl.ds(r0, BLK)])


_sc_agg = pl.kernel(
    _sc_agg_body,
    out_type=jax.ShapeDtypeStruct((NC, N, D), jnp.float32),
    mesh=plsc.VectorSubcoreMesh(
        core_axis_name="c", subcore_axis_name="s",
        num_cores=NC, num_subcores=NS),
    scratch_types=[
        pltpu.VMEM((NCH, CHUNK), jnp.int32),
        pltpu.VMEM((EPW,), jnp.float32),
        pltpu.VMEM((CHUNK,), jnp.int32),
        pltpu.VMEM((CHUNK,), jnp.int32),
        pltpu.VMEM((CHUNK,), jnp.int32),
        pltpu.VMEM((CHUNK,), jnp.int32),
        pltpu.VMEM((CHUNK, D), jnp.float32),
        pltpu.VMEM((CHUNK, D), jnp.float32),
        pltpu.VMEM_SHARED((N, D), jnp.float32),
        pltpu.SemaphoreType.DMA,
        pltpu.SemaphoreType.DMA,
        pltpu.SemaphoreType.DMA,
        pltpu.SemaphoreType.DMA,
    ],
)


_ROW_BLK = 1000
_GRID = N // _ROW_BLK


def _tc_in_body(x_ref, w_ref, b_ref, o_ref):
    o_ref[...] = jnp.maximum(
        jnp.dot(x_ref[...], w_ref[...], preferred_element_type=jnp.float32)
        + b_ref[...], 0.0)


def _tc_in(x, W, b):
    return pl.pallas_call(
        _tc_in_body,
        grid=(_GRID,),
        in_specs=[
            pl.BlockSpec((_ROW_BLK, D), lambda i: (i, 0)),
            pl.BlockSpec((D, D), lambda i: (0, 0)),
            pl.BlockSpec((1, D), lambda i: (0, 0)),
        ],
        out_specs=pl.BlockSpec((_ROW_BLK, D), lambda i: (i, 0)),
        out_shape=jax.ShapeDtypeStruct((N, D), jnp.float32),
    )(x, W, b)


def _tc_mm_body(p_ref, h_ref, wr_ref, br_ref, wo_ref, t_ref, s_ref, q_ref):
    agg = p_ref[0] + p_ref[1]
    t = (jnp.dot(agg, wr_ref[...], preferred_element_type=jnp.float32)
         + jnp.dot(h_ref[...], wo_ref[...], preferred_element_type=jnp.float32)
         + br_ref[...])
    t_ref[...] = t

    @pl.when(pl.program_id(0) == 0)
    def _():
        s_ref[...] = jnp.zeros_like(s_ref)
        q_ref[...] = jnp.zeros_like(q_ref)

    s_ref[...] += jnp.sum(t, axis=0, keepdims=True)
    q_ref[...] += jnp.sum(t * t, axis=0, keepdims=True)


def _tc_mm(parts, h, W_rel, b_rel, W_root):
    return pl.pallas_call(
        _tc_mm_body,
        grid=(_GRID,),
        in_specs=[
            pl.BlockSpec((NC, _ROW_BLK, D), lambda i: (0, i, 0)),
            pl.BlockSpec((_ROW_BLK, D), lambda i: (i, 0)),
            pl.BlockSpec((D, D), lambda i: (0, 0)),
            pl.BlockSpec((1, D), lambda i: (0, 0)),
            pl.BlockSpec((D, D), lambda i: (0, 0)),
        ],
        out_specs=[
            pl.BlockSpec((_ROW_BLK, D), lambda i: (i, 0)),
            pl.BlockSpec((1, D), lambda i: (0, 0)),
            pl.BlockSpec((1, D), lambda i: (0, 0)),
        ],
        out_shape=[
            jax.ShapeDtypeStruct((N, D), jnp.float32),
            jax.ShapeDtypeStruct((1, D), jnp.float32),
            jax.ShapeDtypeStruct((1, D), jnp.float32),
        ],
    )(parts, h, W_rel, b_rel, W_root)


def _tc_bn_body(t_ref, s_ref, q_ref, g_ref, b_ref, f_ref, o_ref):
    mean = s_ref[...] / N
    var = q_ref[...] / N - mean * mean
    scale = g_ref[...] * lax.rsqrt(var + EPS)
    y = (t_ref[...] - mean) * scale + b_ref[...]
    o_ref[...] = jnp.where(f_ref[0, 0] > 0, jnp.maximum(y, 0.0), y)


def _tc_bn(t, ssum, sqsum, gamma, beta, relu_flag):
    return pl.pallas_call(
        _tc_bn_body,
        grid=(_GRID,),
        in_specs=[
            pl.BlockSpec((_ROW_BLK, D), lambda i: (i, 0)),
            pl.BlockSpec((1, D), lambda i: (0, 0)),
            pl.BlockSpec((1, D), lambda i: (0, 0)),
            pl.BlockSpec((1, D), lambda i: (0, 0)),
            pl.BlockSpec((1, D), lambda i: (0, 0)),
            pl.BlockSpec((1, 1), lambda i: (0, 0)),
        ],
        out_specs=pl.BlockSpec((_ROW_BLK, D), lambda i: (i, 0)),
        out_shape=jax.ShapeDtypeStruct((N, D), jnp.float32),
    )(t, ssum, sqsum, gamma, beta, relu_flag)


def kernel(x, adj, features, W_in, b_in, W_rel1, b_rel1, W_root1,
           W_rel2, b_rel2, W_root2, gamma1, beta1):
    comb = ((adj[1] << 14) | adj[0]).reshape(NW, NCH, CHUNK)
    ew = features.reshape(NW, EPW)
    b_in_r = b_in.reshape(1, D)
    gamma_r = gamma1.reshape(1, D)
    beta_r = beta1.reshape(1, D)

    h = _tc_in(x, W_in, b_in_r)

    W_rels = jnp.stack([W_rel1, W_rel2])
    b_rels = jnp.stack([b_rel1.reshape(1, D), b_rel2.reshape(1, D)])
    W_roots = jnp.stack([W_root1, W_root2])
    flags = jnp.array([[[1.0]], [[0.0]]], jnp.float32)

    n_layers = lax.optimization_barrier(jnp.int32(2))

    def _cond(carry):
        i, _ = carry
        return i < n_layers

    def _layer(carry):
        i, hc = carry
        wr = lax.dynamic_index_in_dim(W_rels, i, keepdims=False)
        br = lax.dynamic_index_in_dim(b_rels, i, keepdims=False)
        wroot = lax.dynamic_index_in_dim(W_roots, i, keepdims=False)
        flag = lax.dynamic_index_in_dim(flags, i, keepdims=False)
        parts = _sc_agg(hc, comb, ew)
        t, ssum, qsum = _tc_mm(parts, hc, wr, br, wroot)
        return i + 1, _tc_bn(t, ssum, qsum, gamma_r, beta_r, flag)

    _, out = lax.while_loop(_cond, _layer, (jnp.int32(0), h))
    return out
